# baseline (speedup 1.0000x reference)
import functools

import jax
import jax.numpy as jnp
from jax.experimental import pallas as pl
from jax.experimental.pallas import tpu as pltpu

_INF = 1e30
_BIG_LABEL = 1e9
_K = 5
_LANES = 128
_NSETS = 2


def _knn_body(blk_rows, n_chunks, x_ref, data_ref, labels_ref, mask_ref,
              out_ref, accv_ref, accl_ref):
    i = pl.program_id(0)
    nb = pl.num_programs(0)
    q = x_ref.shape[0]
    cw = blk_rows // n_chunks
    ns = cw // _LANES

    @pl.when(i == 0)
    def _init():
        accv_ref[...] = jnp.full(accv_ref.shape, -_INF, jnp.float32)
        accl_ref[...] = jnp.zeros(accl_ref.shape, jnp.float32)

    xq = x_ref[...]
    half_row = jnp.full((1, xq.shape[1]), 0.5, jnp.float32)
    labf = labels_ref[...].astype(jnp.float32).reshape(1, blk_rows)
    mask = mask_ref[...].reshape(1, blk_rows)

    m = [[accv_ref[:, pl.ds((st * _K + k) * _LANES, _LANES)]
          for k in range(_K)] for st in range(_NSETS)]
    l = [[accl_ref[:, pl.ds((st * _K + k) * _LANES, _LANES)]
          for k in range(_K)] for st in range(_NSETS)]

    for c in range(n_chunks):
        blk_c = data_ref[pl.ds(c * cw, cw), :]
        s_c = jax.lax.dot_general(xq, blk_c, (((1,), (1,)), ((), ())),
                                  preferred_element_type=jnp.float32)
        h_c = jax.lax.dot_general(half_row, blk_c * blk_c,
                                  (((1,), (1,)), ((), ())),
                                  preferred_element_type=jnp.float32)
        h_c = h_c + mask[:, c * cw:(c + 1) * cw]
        mc, lc = m[c % _NSETS], l[c % _NSETS]
        for sl_i in range(ns):
            sl = slice(sl_i * _LANES, (sl_i + 1) * _LANES)
            gsl = slice(c * cw + sl_i * _LANES,
                        c * cw + (sl_i + 1) * _LANES)
            u = s_c[:, sl] - h_c[:, sl]
            lu = jnp.broadcast_to(labf[:, gsl], (q, _LANES))
            for k in range(_K):
                cmp = u > mc[k]
                nm = jnp.where(cmp, u, mc[k])
                nl = jnp.where(cmp, lu, lc[k])
                if k < _K - 1:
                    nu = jnp.where(cmp, mc[k], u)
                    nlu = jnp.where(cmp, lc[k], lu)
                    u, lu = nu, nlu
                mc[k] = nm
                lc[k] = nl
    for st in range(_NSETS):
        for k in range(_K):
            accv_ref[:, pl.ds((st * _K + k) * _LANES, _LANES)] = m[st][k]
            accl_ref[:, pl.ds((st * _K + k) * _LANES, _LANES)] = l[st][k]

    @pl.when(i == nb - 1)
    def _finish():
        cur = accv_ref[...]
        labcur = accl_ref[...]
        vals, labs = [], []
        for _ in range(_K):
            mx = jnp.max(cur, axis=1, keepdims=True)
            eq = cur == mx
            lb = jnp.min(jnp.where(eq, labcur, _BIG_LABEL), axis=1,
                         keepdims=True)
            vals.append(mx)
            labs.append(lb)
            cur = jnp.where(eq, -_INF, cur)
        fv = jnp.concatenate(
            vals + [jnp.full((q, 3), -_INF, jnp.float32)], axis=1)
        fl = jnp.concatenate(
            labs + [jnp.zeros((q, 3), jnp.float32)], axis=1)

        q2 = jnp.sum(xq * xq, axis=1, keepdims=True)
        d2 = jnp.maximum(q2 - 2.0 * fv, 0.0)
        dist = jnp.sqrt(d2)
        w = 1.0 / jnp.maximum(dist, 1e-12)
        w = jnp.where(fv <= -_INF, 0.0, w)
        g_cols = []
        for k in range(8):
            same = fl[:, k:k + 1] == fl
            g_cols.append(jnp.sum(jnp.where(same, w, 0.0), axis=1,
                                  keepdims=True))
        g = jnp.concatenate(g_cols, axis=1)
        gm = jnp.max(g, axis=1, keepdims=True)
        pred = jnp.min(jnp.where(g == gm, fl, _BIG_LABEL), axis=1,
                       keepdims=True)
        cls = jax.lax.broadcasted_iota(jnp.int32, out_ref.shape, 1)
        out_ref[...] = (cls.astype(jnp.float32) == pred).astype(jnp.float32)


def kernel(x, data, labels, classes_mask):
    del classes_mask
    n_keys = data.shape[0]
    q = x.shape[0]
    n_classes = 1000
    blk_rows = 8192
    n_chunks = 8
    nb = pl.cdiv(n_keys, blk_rows)

    col = jnp.arange(nb * blk_rows, dtype=jnp.int32).reshape(nb, 1, blk_rows)
    mask = jnp.where(col < n_keys, 0.0, _INF).astype(jnp.float32)

    body = functools.partial(_knn_body, blk_rows, n_chunks)
    return pl.pallas_call(
        body,
        grid=(nb,),
        in_specs=[
            pl.BlockSpec((q, data.shape[1]), lambda i: (0, 0)),
            pl.BlockSpec((blk_rows, data.shape[1]), lambda i: (i, 0)),
            pl.BlockSpec((blk_rows,), lambda i: (i,)),
            pl.BlockSpec((1, 1, blk_rows), lambda i: (i, 0, 0)),
        ],
        out_specs=pl.BlockSpec((q, n_classes), lambda i: (0, 0)),
        out_shape=jax.ShapeDtypeStruct((q, n_classes), jnp.float32),
        scratch_shapes=[
            pltpu.VMEM((q, _NSETS * _K * _LANES), jnp.float32),
            pltpu.VMEM((q, _NSETS * _K * _LANES), jnp.float32),
        ],
        compiler_params=pltpu.CompilerParams(
            dimension_semantics=("arbitrary",),
        ),
    )(x, data, labels, mask)


# speedup vs baseline: 1.0248x; 1.0248x over previous
import functools

import jax
import jax.numpy as jnp
from jax.experimental import pallas as pl
from jax.experimental.pallas import tpu as pltpu

_INF = 1e30
_BIG_LABEL = 1e9
_K = 5
_LANES = 128
_NSETS = 2


def _knn_body(blk_rows, n_chunks, x_ref, data_ref, labels_ref, mask_ref,
              out_ref, accv_ref, accl_ref):
    i = pl.program_id(0)
    nb = pl.num_programs(0)
    q = x_ref.shape[0]
    cw = blk_rows // n_chunks
    ns = cw // _LANES

    @pl.when(i == 0)
    def _init():
        accv_ref[...] = jnp.full(accv_ref.shape, -_INF, jnp.float32)
        accl_ref[...] = jnp.zeros(accl_ref.shape, jnp.float32)

    xq = x_ref[...]
    half_row = jnp.full((1, xq.shape[1]), 0.5, jnp.float32)
    labf = labels_ref[...].astype(jnp.float32).reshape(1, blk_rows)
    mask = mask_ref[...].reshape(1, blk_rows)

    m = [[accv_ref[:, pl.ds((st * _K + k) * _LANES, _LANES)]
          for k in range(_K)] for st in range(_NSETS)]
    l = [[accl_ref[:, pl.ds((st * _K + k) * _LANES, _LANES)]
          for k in range(_K)] for st in range(_NSETS)]

    for c in range(n_chunks):
        blk_c = data_ref[pl.ds(c * cw, cw), :]
        s_c = jax.lax.dot_general(xq, blk_c, (((1,), (1,)), ((), ())),
                                  preferred_element_type=jnp.float32)
        h_c = jax.lax.dot_general(half_row, blk_c * blk_c,
                                  (((1,), (1,)), ((), ())),
                                  preferred_element_type=jnp.float32)
        h_c = h_c + mask[:, c * cw:(c + 1) * cw]
        mc, lc = m[c % _NSETS], l[c % _NSETS]
        for sl_i in range(ns):
            sl = slice(sl_i * _LANES, (sl_i + 1) * _LANES)
            gsl = slice(c * cw + sl_i * _LANES,
                        c * cw + (sl_i + 1) * _LANES)
            u = s_c[:, sl] - h_c[:, sl]
            lu = jnp.broadcast_to(labf[:, gsl], (q, _LANES))
            for k in range(_K):
                cmp = u > mc[k]
                nm = jnp.where(cmp, u, mc[k])
                nl = jnp.where(cmp, lu, lc[k])
                if k < _K - 1:
                    nu = jnp.where(cmp, mc[k], u)
                    nlu = jnp.where(cmp, lc[k], lu)
                    u, lu = nu, nlu
                mc[k] = nm
                lc[k] = nl
    for st in range(_NSETS):
        for k in range(_K):
            accv_ref[:, pl.ds((st * _K + k) * _LANES, _LANES)] = m[st][k]
            accl_ref[:, pl.ds((st * _K + k) * _LANES, _LANES)] = l[st][k]

    @pl.when(i == nb - 1)
    def _finish():
        cur = accv_ref[...]
        labcur = accl_ref[...]
        vals, labs = [], []
        for _ in range(_K):
            mx = jnp.max(cur, axis=1, keepdims=True)
            eq = cur == mx
            lb = jnp.min(jnp.where(eq, labcur, _BIG_LABEL), axis=1,
                         keepdims=True)
            vals.append(mx)
            labs.append(lb)
            cur = jnp.where(eq, -_INF, cur)
        fv = jnp.concatenate(
            vals + [jnp.full((q, 3), -_INF, jnp.float32)], axis=1)
        fl = jnp.concatenate(
            labs + [jnp.zeros((q, 3), jnp.float32)], axis=1)

        q2 = jnp.sum(xq * xq, axis=1, keepdims=True)
        d2 = jnp.maximum(q2 - 2.0 * fv, 0.0)
        dist = jnp.sqrt(d2)
        w = 1.0 / jnp.maximum(dist, 1e-12)
        w = jnp.where(fv <= -_INF, 0.0, w)
        g_cols = []
        for k in range(8):
            same = fl[:, k:k + 1] == fl
            g_cols.append(jnp.sum(jnp.where(same, w, 0.0), axis=1,
                                  keepdims=True))
        g = jnp.concatenate(g_cols, axis=1)
        gm = jnp.max(g, axis=1, keepdims=True)
        pred = jnp.min(jnp.where(g == gm, fl, _BIG_LABEL), axis=1,
                       keepdims=True)
        cls = jax.lax.broadcasted_iota(jnp.int32, out_ref.shape, 1)
        out_ref[...] = (cls.astype(jnp.float32) == pred).astype(jnp.float32)


def kernel(x, data, labels, classes_mask):
    del classes_mask
    n_keys = data.shape[0]
    q = x.shape[0]
    n_classes = 1000
    blk_rows = 16384
    n_chunks = 16
    nb = pl.cdiv(n_keys, blk_rows)

    col = jnp.arange(nb * blk_rows, dtype=jnp.int32).reshape(nb, 1, blk_rows)
    mask = jnp.where(col < n_keys, 0.0, _INF).astype(jnp.float32)

    body = functools.partial(_knn_body, blk_rows, n_chunks)
    return pl.pallas_call(
        body,
        grid=(nb,),
        in_specs=[
            pl.BlockSpec((q, data.shape[1]), lambda i: (0, 0)),
            pl.BlockSpec((blk_rows, data.shape[1]), lambda i: (i, 0)),
            pl.BlockSpec((blk_rows,), lambda i: (i,)),
            pl.BlockSpec((1, 1, blk_rows), lambda i: (i, 0, 0)),
        ],
        out_specs=pl.BlockSpec((q, n_classes), lambda i: (0, 0)),
        out_shape=jax.ShapeDtypeStruct((q, n_classes), jnp.float32),
        scratch_shapes=[
            pltpu.VMEM((q, _NSETS * _K * _LANES), jnp.float32),
            pltpu.VMEM((q, _NSETS * _K * _LANES), jnp.float32),
        ],
        compiler_params=pltpu.CompilerParams(
            dimension_semantics=("arbitrary",),
        ),
    )(x, data, labels, mask)


# speedup vs baseline: 1.0249x; 1.0001x over previous
import functools

import jax
import jax.numpy as jnp
from jax.experimental import pallas as pl
from jax.experimental.pallas import tpu as pltpu

_INF = 1e30
_BIG_LABEL = 1e9
_K = 5
_LANES = 128
_NSETS = 4


def _knn_body(blk_rows, n_chunks, x_ref, data_ref, labels_ref, mask_ref,
              out_ref, accv_ref, accl_ref):
    i = pl.program_id(0)
    nb = pl.num_programs(0)
    q = x_ref.shape[0]
    cw = blk_rows // n_chunks
    ns = cw // _LANES

    @pl.when(i == 0)
    def _init():
        accv_ref[...] = jnp.full(accv_ref.shape, -_INF, jnp.float32)
        accl_ref[...] = jnp.zeros(accl_ref.shape, jnp.float32)

    xq = x_ref[...]
    half_row = jnp.full((1, xq.shape[1]), 0.5, jnp.float32)
    labf = labels_ref[...].astype(jnp.float32).reshape(1, blk_rows)
    mask = mask_ref[...].reshape(1, blk_rows)

    m = [[accv_ref[:, pl.ds((st * _K + k) * _LANES, _LANES)]
          for k in range(_K)] for st in range(_NSETS)]
    l = [[accl_ref[:, pl.ds((st * _K + k) * _LANES, _LANES)]
          for k in range(_K)] for st in range(_NSETS)]

    for c in range(n_chunks):
        blk_c = data_ref[pl.ds(c * cw, cw), :]
        s_c = jax.lax.dot_general(xq, blk_c, (((1,), (1,)), ((), ())),
                                  preferred_element_type=jnp.float32)
        h_c = jax.lax.dot_general(half_row, blk_c * blk_c,
                                  (((1,), (1,)), ((), ())),
                                  preferred_element_type=jnp.float32)
        h_c = h_c + mask[:, c * cw:(c + 1) * cw]
        mc, lc = m[c % _NSETS], l[c % _NSETS]
        for sl_i in range(ns):
            sl = slice(sl_i * _LANES, (sl_i + 1) * _LANES)
            gsl = slice(c * cw + sl_i * _LANES,
                        c * cw + (sl_i + 1) * _LANES)
            u = s_c[:, sl] - h_c[:, sl]
            lu = jnp.broadcast_to(labf[:, gsl], (q, _LANES))
            for k in range(_K):
                cmp = u > mc[k]
                nm = jnp.where(cmp, u, mc[k])
                nl = jnp.where(cmp, lu, lc[k])
                if k < _K - 1:
                    nu = jnp.where(cmp, mc[k], u)
                    nlu = jnp.where(cmp, lc[k], lu)
                    u, lu = nu, nlu
                mc[k] = nm
                lc[k] = nl
    for st in range(_NSETS):
        for k in range(_K):
            accv_ref[:, pl.ds((st * _K + k) * _LANES, _LANES)] = m[st][k]
            accl_ref[:, pl.ds((st * _K + k) * _LANES, _LANES)] = l[st][k]

    @pl.when(i == nb - 1)
    def _finish():
        cur = accv_ref[...]
        labcur = accl_ref[...]
        vals, labs = [], []
        for _ in range(_K):
            mx = jnp.max(cur, axis=1, keepdims=True)
            eq = cur == mx
            lb = jnp.min(jnp.where(eq, labcur, _BIG_LABEL), axis=1,
                         keepdims=True)
            vals.append(mx)
            labs.append(lb)
            cur = jnp.where(eq, -_INF, cur)
        fv = jnp.concatenate(
            vals + [jnp.full((q, 3), -_INF, jnp.float32)], axis=1)
        fl = jnp.concatenate(
            labs + [jnp.zeros((q, 3), jnp.float32)], axis=1)

        q2 = jnp.sum(xq * xq, axis=1, keepdims=True)
        d2 = jnp.maximum(q2 - 2.0 * fv, 0.0)
        dist = jnp.sqrt(d2)
        w = 1.0 / jnp.maximum(dist, 1e-12)
        w = jnp.where(fv <= -_INF, 0.0, w)
        g_cols = []
        for k in range(8):
            same = fl[:, k:k + 1] == fl
            g_cols.append(jnp.sum(jnp.where(same, w, 0.0), axis=1,
                                  keepdims=True))
        g = jnp.concatenate(g_cols, axis=1)
        gm = jnp.max(g, axis=1, keepdims=True)
        pred = jnp.min(jnp.where(g == gm, fl, _BIG_LABEL), axis=1,
                       keepdims=True)
        cls = jax.lax.broadcasted_iota(jnp.int32, out_ref.shape, 1)
        out_ref[...] = (cls.astype(jnp.float32) == pred).astype(jnp.float32)


def kernel(x, data, labels, classes_mask):
    del classes_mask
    n_keys = data.shape[0]
    q = x.shape[0]
    n_classes = 1000
    blk_rows = 16384
    n_chunks = 16
    nb = pl.cdiv(n_keys, blk_rows)

    col = jnp.arange(nb * blk_rows, dtype=jnp.int32).reshape(nb, 1, blk_rows)
    mask = jnp.where(col < n_keys, 0.0, _INF).astype(jnp.float32)

    body = functools.partial(_knn_body, blk_rows, n_chunks)
    return pl.pallas_call(
        body,
        grid=(nb,),
        in_specs=[
            pl.BlockSpec((q, data.shape[1]), lambda i: (0, 0)),
            pl.BlockSpec((blk_rows, data.shape[1]), lambda i: (i, 0)),
            pl.BlockSpec((blk_rows,), lambda i: (i,)),
            pl.BlockSpec((1, 1, blk_rows), lambda i: (i, 0, 0)),
        ],
        out_specs=pl.BlockSpec((q, n_classes), lambda i: (0, 0)),
        out_shape=jax.ShapeDtypeStruct((q, n_classes), jnp.float32),
        scratch_shapes=[
            pltpu.VMEM((q, _NSETS * _K * _LANES), jnp.float32),
            pltpu.VMEM((q, _NSETS * _K * _LANES), jnp.float32),
        ],
        compiler_params=pltpu.CompilerParams(
            dimension_semantics=("arbitrary",),
        ),
    )(x, data, labels, mask)
